# Initial kernel scaffold; baseline (speedup 1.0000x reference)
import jax
import jax.numpy as jnp
from jax.experimental import pallas as pl


def kernel(heatmap2d, paf2d):
    raise NotImplementedError("write your pallas kernel here")


# speedup vs baseline: 3.0385x; 3.0385x over previous
import jax
import jax.numpy as jnp
from jax.experimental import pallas as pl


_H = 256
_W = 256


def _fused_kernel(hm_ref, paf_ref, out_ref):
    t = hm_ref[0, 0]
    t = jnp.where(t < 0.1, 0.0, t)

    zrow = jnp.zeros((1, _W), dtype=t.dtype)
    zcol = jnp.zeros((_H, 1), dtype=t.dtype)
    nxt_col = jnp.concatenate([t[:, 1:], zcol], axis=1)
    prv_col = jnp.concatenate([zcol, t[:, :-1]], axis=1)
    nxt_row = jnp.concatenate([t[1:, :], zrow], axis=0)
    prv_row = jnp.concatenate([zrow, t[:-1, :]], axis=0)

    peak = (t > nxt_col) & (t > prv_col) & (t > nxt_row) & (t > prv_row)
    out_ref[0, 0, 0] = jnp.where(peak, t, 0.0)

    px = paf_ref[0, 0, 0]
    py = paf_ref[0, 0, 1]
    out_ref[0, 1, 0] = jnp.sqrt(px * px + py * py + 1e-12)


def kernel(heatmap2d, paf2d):
    B, K, H, W = heatmap2d.shape
    paf = paf2d.reshape(B, K, 2, H, W)

    out = pl.pallas_call(
        _fused_kernel,
        grid=(B, K),
        in_specs=[
            pl.BlockSpec((1, 1, H, W), lambda b, k: (b, k, 0, 0)),
            pl.BlockSpec((1, 1, 2, H, W), lambda b, k: (b, k, 0, 0, 0)),
        ],
        out_specs=pl.BlockSpec((1, 2, 1, H, W), lambda b, k: (b, 0, k, 0, 0)),
        out_shape=jax.ShapeDtypeStruct((B, 2, K, H, W), heatmap2d.dtype),
    )(heatmap2d, paf)

    return out.reshape(B, 2 * K, H, W)


# speedup vs baseline: 6.0076x; 1.9771x over previous
import jax
import jax.numpy as jnp
from jax.experimental import pallas as pl


_H = 256
_W = 256


_KC = 19


def _fused_kernel(hm_ref, paf_ref, out_ref):
    t = hm_ref[0]
    t = jnp.where(t < 0.1, 0.0, t)

    kc = t.shape[0]
    zrow = jnp.zeros((kc, 1, _W), dtype=t.dtype)
    zcol = jnp.zeros((kc, _H, 1), dtype=t.dtype)
    nxt_col = jnp.concatenate([t[:, :, 1:], zcol], axis=2)
    prv_col = jnp.concatenate([zcol, t[:, :, :-1]], axis=2)
    nxt_row = jnp.concatenate([t[:, 1:, :], zrow], axis=1)
    prv_row = jnp.concatenate([zrow, t[:, :-1, :]], axis=1)

    peak = (t > nxt_col) & (t > prv_col) & (t > nxt_row) & (t > prv_row)
    out_ref[0, 0] = jnp.where(peak, t, 0.0)

    px = paf_ref[0, :, 0]
    py = paf_ref[0, :, 1]
    out_ref[0, 1] = jnp.sqrt(px * px + py * py + 1e-12)


def kernel(heatmap2d, paf2d):
    B, K, H, W = heatmap2d.shape
    paf = paf2d.reshape(B, K, 2, H, W)

    out = pl.pallas_call(
        _fused_kernel,
        grid=(B, K // _KC),
        in_specs=[
            pl.BlockSpec((1, _KC, H, W), lambda b, k: (b, k, 0, 0)),
            pl.BlockSpec((1, _KC, 2, H, W), lambda b, k: (b, k, 0, 0, 0)),
        ],
        out_specs=pl.BlockSpec((1, 2, _KC, H, W), lambda b, k: (b, 0, k, 0, 0)),
        out_shape=jax.ShapeDtypeStruct((B, 2, K, H, W), heatmap2d.dtype),
    )(heatmap2d, paf)

    return out.reshape(B, 2 * K, H, W)
